# Initial kernel scaffold; baseline (speedup 1.0000x reference)
import jax
import jax.numpy as jnp
from jax.experimental import pallas as pl


def kernel(query, memory_keys, memory_values):
    raise NotImplementedError("write your pallas kernel here")


# speedup vs baseline: 1.9546x; 1.9546x over previous
import functools

import jax
import jax.numpy as jnp
from jax import lax
from jax.experimental import pallas as pl
from jax.experimental.pallas import tpu as pltpu
from jax.experimental.pallas import tpu_sc as plsc

B = 1024
D = 32
CAP = 100000
K = 5
BLK = 2048
CAP_PAD = 100352
NBLK = CAP_PAD // BLK
BIGI = 2**30


def _topk_body(q_ref, k_ref, oi_ref, rv_ref, ri_ref):
    i = pl.program_id(0)

    @pl.when(i == 0)
    def _init():
        rv_ref[...] = jnp.full((B, 128), -jnp.inf, jnp.float32)
        ri_ref[...] = jnp.full((B, 128), BIGI, jnp.int32)

    q = q_ref[...]
    qn = q / jnp.maximum(jnp.sqrt(jnp.sum(q * q, axis=1, keepdims=True)), 1e-12)
    kb = k_ref[...]
    kn = kb / jnp.maximum(jnp.sqrt(jnp.sum(kb * kb, axis=1, keepdims=True)), 1e-12)
    s = lax.dot_general(qn, kn, (((1,), (1,)), ((), ())),
                        preferred_element_type=jnp.float32)
    giota = lax.broadcasted_iota(jnp.int32, (B, BLK), 1) + i * BLK
    s = jnp.where(giota < CAP, s, -jnp.inf)

    rv = rv_ref[...]
    ri = ri_ref[...]
    newv = []
    newi = []
    for _ in range(K):
        ms = jnp.max(s, axis=1, keepdims=True)
        mr = jnp.max(rv, axis=1, keepdims=True)
        m = jnp.maximum(ms, mr)
        is_ = jnp.min(jnp.where(s == m, giota, BIGI), axis=1, keepdims=True)
        ir_ = jnp.min(jnp.where(rv == m, ri, BIGI), axis=1, keepdims=True)
        idx = jnp.minimum(is_, ir_)
        newv.append(m)
        newi.append(idx)
        s = jnp.where(giota == idx, -jnp.inf, s)
        rv = jnp.where(ri == idx, -jnp.inf, rv)
    nv = jnp.concatenate(newv + [jnp.full((B, 128 - K), -jnp.inf, jnp.float32)], axis=1)
    ni = jnp.concatenate(newi + [jnp.full((B, 128 - K), BIGI, jnp.int32)], axis=1)
    rv_ref[...] = nv
    ri_ref[...] = ni
    oi_ref[...] = ni


def _topk_indices(query, keys):
    keys_p = jnp.pad(keys, ((0, CAP_PAD - CAP), (0, 0)))
    out = pl.pallas_call(
        _topk_body,
        grid=(NBLK,),
        in_specs=[
            pl.BlockSpec((B, D), lambda i: (0, 0)),
            pl.BlockSpec((BLK, D), lambda i: (i, 0)),
        ],
        out_specs=pl.BlockSpec((B, 128), lambda i: (0, 0)),
        out_shape=jax.ShapeDtypeStruct((B, 128), jnp.int32),
        scratch_shapes=[
            pltpu.VMEM((B, 128), jnp.float32),
            pltpu.VMEM((B, 128), jnp.int32),
        ],
    )(query, keys_p)
    return out[:, :K]


_NC = 2
_NS = 16
_NW = _NC * _NS
_QPW = B // _NW
_RPW = _QPW * K


def _gather_mean_body(table_hbm, idx_hbm, out_hbm, idx_v, rows_v, mean_v, sem):
    wid = lax.axis_index("s") * _NC + lax.axis_index("c")
    base = wid * _RPW
    pltpu.sync_copy(idx_hbm.at[pl.ds(base, _RPW)], idx_v)
    pltpu.async_copy(table_hbm.at[idx_v], rows_v, sem).wait()
    for q in range(_QPW):
        for h in range(D // 16):
            acc = rows_v[q * K + 0, pl.ds(h * 16, 16)]
            for j in range(1, K):
                acc = acc + rows_v[q * K + j, pl.ds(h * 16, 16)]
            mean_v[q, pl.ds(h * 16, 16)] = acc * jnp.float32(1.0 / K)
    pltpu.sync_copy(mean_v, out_hbm.at[pl.ds(wid * _QPW, _QPW)])


@functools.cache
def _gather_mean():
    return pl.kernel(
        _gather_mean_body,
        mesh=plsc.VectorSubcoreMesh(core_axis_name="c", subcore_axis_name="s"),
        out_type=jax.ShapeDtypeStruct((B, D), jnp.float32),
        scratch_types=[
            pltpu.VMEM((_RPW,), jnp.int32),
            pltpu.VMEM((_RPW, D), jnp.float32),
            pltpu.VMEM((_QPW, D), jnp.float32),
            pltpu.SemaphoreType.DMA,
        ],
        compiler_params=pltpu.CompilerParams(use_tc_tiling_on_sc=False),
    )


def kernel(query, memory_keys, memory_values):
    if query.ndim == 3:
        query = query.mean(axis=1)
    top_idx = _topk_indices(query, memory_keys)
    flat_idx = top_idx.reshape(B * K)
    return _gather_mean()(memory_values, flat_idx)


# speedup vs baseline: 3.2385x; 1.6568x over previous
import functools

import jax
import jax.numpy as jnp
from jax import lax
from jax.experimental import pallas as pl
from jax.experimental.pallas import tpu as pltpu
from jax.experimental.pallas import tpu_sc as plsc

B = 1024
D = 32
CAP = 100000
K = 5
BLK = 2048
CAP_PAD = 100352
NBLK = CAP_PAD // BLK
BIGI = 2**30


def _topk_body_fast(q_ref, k_ref, oi_ref, om_ref, rv_ref, ri_ref, m3_ref):
    i = pl.program_id(0)

    @pl.when(i == 0)
    def _init():
        rv_ref[...] = jnp.full((B, 128), -jnp.inf, jnp.float32)
        ri_ref[...] = jnp.full((B, 128), BIGI, jnp.int32)
        m3_ref[...] = jnp.full((B, 128), -jnp.inf, jnp.float32)

    q = q_ref[...]
    qn = q / jnp.maximum(jnp.sqrt(jnp.sum(q * q, axis=1, keepdims=True)), 1e-12)
    kb = k_ref[...]
    kn = kb / jnp.maximum(jnp.sqrt(jnp.sum(kb * kb, axis=1, keepdims=True)), 1e-12)
    s = lax.dot_general(qn, kn, (((1,), (1,)), ((), ())),
                        preferred_element_type=jnp.float32)

    liota = lax.broadcasted_iota(jnp.int32, (B, 128), 1)
    v1 = i1 = v2 = i2 = v3 = None
    for k in range(BLK // 128):
        base = i * BLK + k * 128
        ix = liota + base
        x = jnp.where(ix < CAP, s[:, k * 128:(k + 1) * 128], -jnp.inf)
        if k == 0:
            v1, i1 = x, ix
            v2 = jnp.full((B, 128), -jnp.inf, jnp.float32)
            i2 = jnp.full((B, 128), BIGI, jnp.int32)
            v3 = jnp.full((B, 128), -jnp.inf, jnp.float32)
        else:
            gt1 = x > v1
            gt2 = x > v2
            gt3 = x > v3
            v3 = jnp.where(gt2, v2, jnp.where(gt3, x, v3))
            v2 = jnp.where(gt1, v1, jnp.where(gt2, x, v2))
            i2 = jnp.where(gt1, i1, jnp.where(gt2, ix, i2))
            v1 = jnp.where(gt1, x, v1)
            i1 = jnp.where(gt1, ix, i1)
    m3_ref[...] = jnp.maximum(
        m3_ref[...], jnp.max(v3, axis=1, keepdims=True))

    c = jnp.concatenate([v1, v2, rv_ref[...]], axis=1)
    ci = jnp.concatenate([i1, i2, ri_ref[...]], axis=1)
    newv = []
    newi = []
    for _ in range(K):
        m = jnp.max(c, axis=1, keepdims=True)
        idx = jnp.min(jnp.where(c == m, ci, BIGI), axis=1, keepdims=True)
        newv.append(m)
        newi.append(idx)
        c = jnp.where(ci == idx, -jnp.inf, c)
    nv = jnp.concatenate(newv + [jnp.full((B, 128 - K), -jnp.inf, jnp.float32)], axis=1)
    ni = jnp.concatenate(newi + [jnp.full((B, 128 - K), BIGI, jnp.int32)], axis=1)
    rv_ref[...] = nv
    ri_ref[...] = ni
    oi_ref[...] = ni
    om_ref[...] = jnp.concatenate(
        [newv[K - 1], m3_ref[:, :1], jnp.zeros((B, 126), jnp.float32)], axis=1)


def _topk_fast(query, keys):
    oi, om = pl.pallas_call(
        _topk_body_fast,
        grid=(NBLK,),
        in_specs=[
            pl.BlockSpec((B, D), lambda i: (0, 0)),
            pl.BlockSpec((BLK, D), lambda i: (i, 0)),
        ],
        out_specs=[
            pl.BlockSpec((B, 128), lambda i: (0, 0)),
            pl.BlockSpec((B, 128), lambda i: (0, 0)),
        ],
        out_shape=[
            jax.ShapeDtypeStruct((B, 128), jnp.int32),
            jax.ShapeDtypeStruct((B, 128), jnp.float32),
        ],
        scratch_shapes=[
            pltpu.VMEM((B, 128), jnp.float32),
            pltpu.VMEM((B, 128), jnp.int32),
            pltpu.VMEM((B, 128), jnp.float32),
        ],
    )(query, keys)
    t5 = om[:, 0]
    m3 = om[:, 1]
    ok = jnp.all(m3 < t5)
    return oi[:, :K], ok


def _topk_body(q_ref, k_ref, oi_ref, rv_ref, ri_ref):
    i = pl.program_id(0)

    @pl.when(i == 0)
    def _init():
        rv_ref[...] = jnp.full((B, 128), -jnp.inf, jnp.float32)
        ri_ref[...] = jnp.full((B, 128), BIGI, jnp.int32)

    q = q_ref[...]
    qn = q / jnp.maximum(jnp.sqrt(jnp.sum(q * q, axis=1, keepdims=True)), 1e-12)
    kb = k_ref[...]
    kn = kb / jnp.maximum(jnp.sqrt(jnp.sum(kb * kb, axis=1, keepdims=True)), 1e-12)
    s = lax.dot_general(qn, kn, (((1,), (1,)), ((), ())),
                        preferred_element_type=jnp.float32)
    giota = lax.broadcasted_iota(jnp.int32, (B, BLK), 1) + i * BLK
    s = jnp.where(giota < CAP, s, -jnp.inf)

    rv = rv_ref[...]
    ri = ri_ref[...]
    newv = []
    newi = []
    for _ in range(K):
        ms = jnp.max(s, axis=1, keepdims=True)
        mr = jnp.max(rv, axis=1, keepdims=True)
        m = jnp.maximum(ms, mr)
        is_ = jnp.min(jnp.where(s == m, giota, BIGI), axis=1, keepdims=True)
        ir_ = jnp.min(jnp.where(rv == m, ri, BIGI), axis=1, keepdims=True)
        idx = jnp.minimum(is_, ir_)
        newv.append(m)
        newi.append(idx)
        s = jnp.where(giota == idx, -jnp.inf, s)
        rv = jnp.where(ri == idx, -jnp.inf, rv)
    nv = jnp.concatenate(newv + [jnp.full((B, 128 - K), -jnp.inf, jnp.float32)], axis=1)
    ni = jnp.concatenate(newi + [jnp.full((B, 128 - K), BIGI, jnp.int32)], axis=1)
    rv_ref[...] = nv
    ri_ref[...] = ni
    oi_ref[...] = ni


def _topk_indices(query, keys):
    keys_p = jnp.pad(keys, ((0, CAP_PAD - CAP), (0, 0)))
    out = pl.pallas_call(
        _topk_body,
        grid=(NBLK,),
        in_specs=[
            pl.BlockSpec((B, D), lambda i: (0, 0)),
            pl.BlockSpec((BLK, D), lambda i: (i, 0)),
        ],
        out_specs=pl.BlockSpec((B, 128), lambda i: (0, 0)),
        out_shape=jax.ShapeDtypeStruct((B, 128), jnp.int32),
        scratch_shapes=[
            pltpu.VMEM((B, 128), jnp.float32),
            pltpu.VMEM((B, 128), jnp.int32),
        ],
    )(query, keys_p)
    return out[:, :K]


_NC = 2
_NS = 16
_NW = _NC * _NS
_QPW = B // _NW
_RPW = _QPW * K


def _gather_mean_body(table_hbm, idx_hbm, out_hbm, idx_v, rows_v, mean_v, sem):
    wid = lax.axis_index("s") * _NC + lax.axis_index("c")
    base = wid * _RPW
    pltpu.sync_copy(idx_hbm.at[pl.ds(base, _RPW)], idx_v)
    pltpu.async_copy(table_hbm.at[idx_v], rows_v, sem).wait()
    for q in range(_QPW):
        for h in range(D // 16):
            acc = rows_v[q * K + 0, pl.ds(h * 16, 16)]
            for j in range(1, K):
                acc = acc + rows_v[q * K + j, pl.ds(h * 16, 16)]
            mean_v[q, pl.ds(h * 16, 16)] = acc * jnp.float32(1.0 / K)
    pltpu.sync_copy(mean_v, out_hbm.at[pl.ds(wid * _QPW, _QPW)])


@functools.cache
def _gather_mean():
    return pl.kernel(
        _gather_mean_body,
        mesh=plsc.VectorSubcoreMesh(core_axis_name="c", subcore_axis_name="s"),
        out_type=jax.ShapeDtypeStruct((B, D), jnp.float32),
        scratch_types=[
            pltpu.VMEM((_RPW,), jnp.int32),
            pltpu.VMEM((_RPW, D), jnp.float32),
            pltpu.VMEM((_QPW, D), jnp.float32),
            pltpu.SemaphoreType.DMA,
        ],
        compiler_params=pltpu.CompilerParams(use_tc_tiling_on_sc=False),
    )


def kernel(query, memory_keys, memory_values):
    if query.ndim == 3:
        query = query.mean(axis=1)
    fast_idx, ok = _topk_fast(query, memory_keys)
    top_idx = lax.cond(ok,
                       lambda: fast_idx,
                       lambda: _topk_indices(query, memory_keys))
    flat_idx = top_idx.reshape(B * K)
    return _gather_mean()(memory_values, flat_idx)


# speedup vs baseline: 3.2461x; 1.0024x over previous
import functools

import jax
import jax.numpy as jnp
from jax import lax
from jax.experimental import pallas as pl
from jax.experimental.pallas import tpu as pltpu
from jax.experimental.pallas import tpu_sc as plsc

B = 1024
D = 32
CAP = 100000
K = 5
BLK = 2048
CAP_PAD = 100352
NBLK = CAP_PAD // BLK
BIGI = 2**30
BIGF = float(2**30)


def _topk_body_fast(q_ref, k_ref, oi_ref, om_ref, rv_ref, ri_ref, m3_ref):
    i = pl.program_id(0)

    @pl.when(i == 0)
    def _init():
        rv_ref[...] = jnp.full((B, 128), -jnp.inf, jnp.float32)
        ri_ref[...] = jnp.full((B, 128), BIGF, jnp.float32)
        m3_ref[...] = jnp.full((B, 128), -jnp.inf, jnp.float32)

    q = q_ref[...]
    qn = q / jnp.maximum(jnp.sqrt(jnp.sum(q * q, axis=1, keepdims=True)), 1e-12)
    kb = k_ref[...]
    kn = kb / jnp.maximum(jnp.sqrt(jnp.sum(kb * kb, axis=1, keepdims=True)), 1e-12)
    s = lax.dot_general(qn, kn, (((1,), (1,)), ((), ())),
                        preferred_element_type=jnp.float32)

    liota = lax.broadcasted_iota(jnp.int32, (B, 128), 1).astype(jnp.float32)
    oob_slice = (CAP - (NBLK - 1) * BLK) // 128
    fbase = (i * BLK).astype(jnp.float32)
    v1 = s1 = v2 = s2 = v3 = None
    for k in range(BLK // 128):
        x = s[:, k * 128:(k + 1) * 128]
        if k >= oob_slice:
            x = jnp.where(liota + (fbase + k * 128) < CAP, x, -jnp.inf)
        kf = jnp.float32(k)
        if k == 0:
            v1 = x
            s1 = jnp.zeros((B, 128), jnp.float32)
            v2 = jnp.full((B, 128), -jnp.inf, jnp.float32)
            s2 = jnp.zeros((B, 128), jnp.float32)
            v3 = jnp.full((B, 128), -jnp.inf, jnp.float32)
        else:
            gt1 = x > v1
            gt2 = x > v2
            gt3 = x > v3
            v3 = jnp.where(gt2, v2, jnp.where(gt3, x, v3))
            v2 = jnp.where(gt1, v1, jnp.where(gt2, x, v2))
            s2 = jnp.where(gt1, s1, jnp.where(gt2, kf, s2))
            v1 = jnp.where(gt1, x, v1)
            s1 = jnp.where(gt1, kf, s1)
    m3_ref[...] = jnp.maximum(
        m3_ref[...], jnp.max(v3, axis=1, keepdims=True))

    gb = liota + fbase
    c = jnp.concatenate([v1, v2, rv_ref[...]], axis=1)
    ci = jnp.concatenate([s1 * 128.0 + gb, s2 * 128.0 + gb, ri_ref[...]], axis=1)
    newv = []
    newi = []
    for _ in range(K):
        m = jnp.max(c, axis=1, keepdims=True)
        idx = jnp.min(jnp.where(c == m, ci, BIGF), axis=1, keepdims=True)
        newv.append(m)
        newi.append(idx)
        c = jnp.where(ci == idx, -jnp.inf, c)
    nv = jnp.concatenate(newv + [jnp.full((B, 128 - K), -jnp.inf, jnp.float32)], axis=1)
    ni = jnp.concatenate(newi + [jnp.full((B, 128 - K), BIGF, jnp.float32)], axis=1)
    rv_ref[...] = nv
    ri_ref[...] = ni
    oi_ref[...] = ni.astype(jnp.int32)
    om_ref[...] = jnp.concatenate(
        [newv[K - 1], m3_ref[:, :1], jnp.zeros((B, 126), jnp.float32)], axis=1)


def _topk_fast(query, keys):
    oi, om = pl.pallas_call(
        _topk_body_fast,
        grid=(NBLK,),
        in_specs=[
            pl.BlockSpec((B, D), lambda i: (0, 0)),
            pl.BlockSpec((BLK, D), lambda i: (i, 0)),
        ],
        out_specs=[
            pl.BlockSpec((B, 128), lambda i: (0, 0)),
            pl.BlockSpec((B, 128), lambda i: (0, 0)),
        ],
        out_shape=[
            jax.ShapeDtypeStruct((B, 128), jnp.int32),
            jax.ShapeDtypeStruct((B, 128), jnp.float32),
        ],
        scratch_shapes=[
            pltpu.VMEM((B, 128), jnp.float32),
            pltpu.VMEM((B, 128), jnp.float32),
            pltpu.VMEM((B, 128), jnp.float32),
        ],
    )(query, keys)
    t5 = om[:, 0]
    m3 = om[:, 1]
    ok = jnp.all(m3 < t5)
    return oi[:, :K], ok


def _topk_body(q_ref, k_ref, oi_ref, rv_ref, ri_ref):
    i = pl.program_id(0)

    @pl.when(i == 0)
    def _init():
        rv_ref[...] = jnp.full((B, 128), -jnp.inf, jnp.float32)
        ri_ref[...] = jnp.full((B, 128), BIGI, jnp.int32)

    q = q_ref[...]
    qn = q / jnp.maximum(jnp.sqrt(jnp.sum(q * q, axis=1, keepdims=True)), 1e-12)
    kb = k_ref[...]
    kn = kb / jnp.maximum(jnp.sqrt(jnp.sum(kb * kb, axis=1, keepdims=True)), 1e-12)
    s = lax.dot_general(qn, kn, (((1,), (1,)), ((), ())),
                        preferred_element_type=jnp.float32)
    giota = lax.broadcasted_iota(jnp.int32, (B, BLK), 1) + i * BLK
    s = jnp.where(giota < CAP, s, -jnp.inf)

    rv = rv_ref[...]
    ri = ri_ref[...]
    newv = []
    newi = []
    for _ in range(K):
        ms = jnp.max(s, axis=1, keepdims=True)
        mr = jnp.max(rv, axis=1, keepdims=True)
        m = jnp.maximum(ms, mr)
        is_ = jnp.min(jnp.where(s == m, giota, BIGI), axis=1, keepdims=True)
        ir_ = jnp.min(jnp.where(rv == m, ri, BIGI), axis=1, keepdims=True)
        idx = jnp.minimum(is_, ir_)
        newv.append(m)
        newi.append(idx)
        s = jnp.where(giota == idx, -jnp.inf, s)
        rv = jnp.where(ri == idx, -jnp.inf, rv)
    nv = jnp.concatenate(newv + [jnp.full((B, 128 - K), -jnp.inf, jnp.float32)], axis=1)
    ni = jnp.concatenate(newi + [jnp.full((B, 128 - K), BIGI, jnp.int32)], axis=1)
    rv_ref[...] = nv
    ri_ref[...] = ni
    oi_ref[...] = ni


def _topk_indices(query, keys):
    keys_p = jnp.pad(keys, ((0, CAP_PAD - CAP), (0, 0)))
    out = pl.pallas_call(
        _topk_body,
        grid=(NBLK,),
        in_specs=[
            pl.BlockSpec((B, D), lambda i: (0, 0)),
            pl.BlockSpec((BLK, D), lambda i: (i, 0)),
        ],
        out_specs=pl.BlockSpec((B, 128), lambda i: (0, 0)),
        out_shape=jax.ShapeDtypeStruct((B, 128), jnp.int32),
        scratch_shapes=[
            pltpu.VMEM((B, 128), jnp.float32),
            pltpu.VMEM((B, 128), jnp.int32),
        ],
    )(query, keys_p)
    return out[:, :K]


_NC = 2
_NS = 16
_NW = _NC * _NS
_QPW = B // _NW
_RPW = _QPW * K


def _gather_mean_body(table_hbm, idx_hbm, out_hbm, idx_v, rows_v, mean_v, sem):
    wid = lax.axis_index("s") * _NC + lax.axis_index("c")
    base = wid * _RPW
    pltpu.sync_copy(idx_hbm.at[pl.ds(base, _RPW)], idx_v)
    pltpu.async_copy(table_hbm.at[idx_v], rows_v, sem).wait()
    for q in range(_QPW):
        for h in range(D // 16):
            acc = rows_v[q * K + 0, pl.ds(h * 16, 16)]
            for j in range(1, K):
                acc = acc + rows_v[q * K + j, pl.ds(h * 16, 16)]
            mean_v[q, pl.ds(h * 16, 16)] = acc * jnp.float32(1.0 / K)
    pltpu.sync_copy(mean_v, out_hbm.at[pl.ds(wid * _QPW, _QPW)])


@functools.cache
def _gather_mean():
    return pl.kernel(
        _gather_mean_body,
        mesh=plsc.VectorSubcoreMesh(core_axis_name="c", subcore_axis_name="s"),
        out_type=jax.ShapeDtypeStruct((B, D), jnp.float32),
        scratch_types=[
            pltpu.VMEM((_RPW,), jnp.int32),
            pltpu.VMEM((_RPW, D), jnp.float32),
            pltpu.VMEM((_QPW, D), jnp.float32),
            pltpu.SemaphoreType.DMA,
        ],
        compiler_params=pltpu.CompilerParams(use_tc_tiling_on_sc=False),
    )


def kernel(query, memory_keys, memory_values):
    if query.ndim == 3:
        query = query.mean(axis=1)
    fast_idx, ok = _topk_fast(query, memory_keys)
    top_idx = lax.cond(ok,
                       lambda: fast_idx,
                       lambda: _topk_indices(query, memory_keys))
    flat_idx = top_idx.reshape(B * K)
    return _gather_mean()(memory_values, flat_idx)


# speedup vs baseline: 4.1591x; 1.2813x over previous
import functools

import jax
import jax.numpy as jnp
from jax import lax
from jax.experimental import pallas as pl
from jax.experimental.pallas import tpu as pltpu
from jax.experimental.pallas import tpu_sc as plsc

B = 1024
D = 32
CAP = 100000
K = 5
BLK = 4096
CAP_PAD = 102400
NBLK = CAP_PAD // BLK
BIGI = 2**30
BIGF = float(2**30)


def _topk_body_fast(q_ref, k_ref, oi_ref, om_ref, rv_ref, ri_ref, m3_ref):
    i = pl.program_id(0)

    @pl.when(i == 0)
    def _init():
        rv_ref[...] = jnp.full((B, 128), -jnp.inf, jnp.float32)
        ri_ref[...] = jnp.full((B, 128), BIGF, jnp.float32)
        m3_ref[...] = jnp.full((B, 128), -jnp.inf, jnp.float32)

    q = q_ref[...]
    qn = q / jnp.maximum(jnp.sqrt(jnp.sum(q * q, axis=1, keepdims=True)), 1e-12)
    kb = k_ref[...]
    kn = kb / jnp.maximum(jnp.sqrt(jnp.sum(kb * kb, axis=1, keepdims=True)), 1e-12)
    s = lax.dot_general(qn, kn, (((1,), (1,)), ((), ())),
                        preferred_element_type=jnp.float32)

    liota = lax.broadcasted_iota(jnp.int32, (B, 128), 1).astype(jnp.float32)
    oob_slice = (CAP - (NBLK - 1) * BLK) // 128
    fbase = (i * BLK).astype(jnp.float32)
    v1 = s1 = v2 = s2 = v3 = None
    for k in range(BLK // 128):
        x = s[:, k * 128:(k + 1) * 128]
        if k >= oob_slice:
            x = jnp.where(liota + (fbase + k * 128) < CAP, x, -jnp.inf)
        kf = jnp.float32(k)
        if k == 0:
            v1 = x
            s1 = jnp.zeros((B, 128), jnp.float32)
            v2 = jnp.full((B, 128), -jnp.inf, jnp.float32)
            s2 = jnp.zeros((B, 128), jnp.float32)
            v3 = jnp.full((B, 128), -jnp.inf, jnp.float32)
        else:
            gt1 = x > v1
            gt2 = x > v2
            gt3 = x > v3
            v3 = jnp.where(gt2, v2, jnp.where(gt3, x, v3))
            v2 = jnp.where(gt1, v1, jnp.where(gt2, x, v2))
            s2 = jnp.where(gt1, s1, jnp.where(gt2, kf, s2))
            v1 = jnp.where(gt1, x, v1)
            s1 = jnp.where(gt1, kf, s1)
    m3_ref[...] = jnp.maximum(
        m3_ref[...], jnp.max(v3, axis=1, keepdims=True))

    gb = liota + fbase
    c = jnp.concatenate([v1, v2, rv_ref[...]], axis=1)
    ci = jnp.concatenate([s1 * 128.0 + gb, s2 * 128.0 + gb, ri_ref[...]], axis=1)
    newv = []
    newi = []
    for _ in range(K):
        m = jnp.max(c, axis=1, keepdims=True)
        idx = jnp.min(jnp.where(c == m, ci, BIGF), axis=1, keepdims=True)
        newv.append(m)
        newi.append(idx)
        c = jnp.where(ci == idx, -jnp.inf, c)
    nv = jnp.concatenate(newv + [jnp.full((B, 128 - K), -jnp.inf, jnp.float32)], axis=1)
    ni = jnp.concatenate(newi + [jnp.full((B, 128 - K), BIGF, jnp.float32)], axis=1)
    rv_ref[...] = nv
    ri_ref[...] = ni
    oi_ref[...] = ni.astype(jnp.int32)
    om_ref[...] = jnp.concatenate(
        [newv[K - 1], m3_ref[:, :1], jnp.zeros((B, 126), jnp.float32)], axis=1)


def _topk_fast(query, keys):
    oi, om = pl.pallas_call(
        _topk_body_fast,
        grid=(NBLK,),
        in_specs=[
            pl.BlockSpec((B, D), lambda i: (0, 0)),
            pl.BlockSpec((BLK, D), lambda i: (i, 0)),
        ],
        out_specs=[
            pl.BlockSpec((B, 128), lambda i: (0, 0)),
            pl.BlockSpec((B, 128), lambda i: (0, 0)),
        ],
        out_shape=[
            jax.ShapeDtypeStruct((B, 128), jnp.int32),
            jax.ShapeDtypeStruct((B, 128), jnp.float32),
        ],
        scratch_shapes=[
            pltpu.VMEM((B, 128), jnp.float32),
            pltpu.VMEM((B, 128), jnp.float32),
            pltpu.VMEM((B, 128), jnp.float32),
        ],
    )(query, keys)
    t5 = om[:, 0]
    m3 = om[:, 1]
    ok = jnp.all(m3 < t5)
    return oi, ok


def _topk_body(q_ref, k_ref, oi_ref, rv_ref, ri_ref):
    i = pl.program_id(0)

    @pl.when(i == 0)
    def _init():
        rv_ref[...] = jnp.full((B, 128), -jnp.inf, jnp.float32)
        ri_ref[...] = jnp.full((B, 128), BIGI, jnp.int32)

    q = q_ref[...]
    qn = q / jnp.maximum(jnp.sqrt(jnp.sum(q * q, axis=1, keepdims=True)), 1e-12)
    kb = k_ref[...]
    kn = kb / jnp.maximum(jnp.sqrt(jnp.sum(kb * kb, axis=1, keepdims=True)), 1e-12)
    s = lax.dot_general(qn, kn, (((1,), (1,)), ((), ())),
                        preferred_element_type=jnp.float32)
    giota = lax.broadcasted_iota(jnp.int32, (B, BLK), 1) + i * BLK
    s = jnp.where(giota < CAP, s, -jnp.inf)

    rv = rv_ref[...]
    ri = ri_ref[...]
    newv = []
    newi = []
    for _ in range(K):
        ms = jnp.max(s, axis=1, keepdims=True)
        mr = jnp.max(rv, axis=1, keepdims=True)
        m = jnp.maximum(ms, mr)
        is_ = jnp.min(jnp.where(s == m, giota, BIGI), axis=1, keepdims=True)
        ir_ = jnp.min(jnp.where(rv == m, ri, BIGI), axis=1, keepdims=True)
        idx = jnp.minimum(is_, ir_)
        newv.append(m)
        newi.append(idx)
        s = jnp.where(giota == idx, -jnp.inf, s)
        rv = jnp.where(ri == idx, -jnp.inf, rv)
    nv = jnp.concatenate(newv + [jnp.full((B, 128 - K), -jnp.inf, jnp.float32)], axis=1)
    ni = jnp.concatenate(newi + [jnp.full((B, 128 - K), BIGI, jnp.int32)], axis=1)
    rv_ref[...] = nv
    ri_ref[...] = ni
    oi_ref[...] = ni


def _topk_indices(query, keys):
    keys_p = jnp.pad(keys, ((0, CAP_PAD - CAP), (0, 0)))
    out = pl.pallas_call(
        _topk_body,
        grid=(NBLK,),
        in_specs=[
            pl.BlockSpec((B, D), lambda i: (0, 0)),
            pl.BlockSpec((BLK, D), lambda i: (i, 0)),
        ],
        out_specs=pl.BlockSpec((B, 128), lambda i: (0, 0)),
        out_shape=jax.ShapeDtypeStruct((B, 128), jnp.int32),
        scratch_shapes=[
            pltpu.VMEM((B, 128), jnp.float32),
            pltpu.VMEM((B, 128), jnp.int32),
        ],
    )(query, keys_p)
    return out


_NC = 2
_NS = 16
_NW = _NC * _NS
_QPW = B // _NW
_RPW = _QPW * K


def _gather_mean_body(table_hbm, idx_hbm, out_hbm, idx_v, rows_v, mean_v, sem):
    wid = lax.axis_index("s") * _NC + lax.axis_index("c")
    base = wid * _RPW
    pltpu.sync_copy(idx_hbm.at[pl.ds(base, _RPW)], idx_v)
    pltpu.async_copy(table_hbm.at[idx_v], rows_v, sem).wait()
    for q in range(_QPW):
        for h in range(D // 16):
            acc = rows_v[q * K + 0, pl.ds(h * 16, 16)]
            for j in range(1, K):
                acc = acc + rows_v[q * K + j, pl.ds(h * 16, 16)]
            mean_v[q, pl.ds(h * 16, 16)] = acc * jnp.float32(1.0 / K)
    pltpu.sync_copy(mean_v, out_hbm.at[pl.ds(wid * _QPW, _QPW)])


@functools.cache
def _gather_mean():
    return pl.kernel(
        _gather_mean_body,
        mesh=plsc.VectorSubcoreMesh(core_axis_name="c", subcore_axis_name="s"),
        out_type=jax.ShapeDtypeStruct((B, D), jnp.float32),
        scratch_types=[
            pltpu.VMEM((_RPW,), jnp.int32),
            pltpu.VMEM((_RPW, D), jnp.float32),
            pltpu.VMEM((_QPW, D), jnp.float32),
            pltpu.SemaphoreType.DMA,
        ],
        compiler_params=pltpu.CompilerParams(use_tc_tiling_on_sc=False),
    )


def kernel(query, memory_keys, memory_values):
    if query.ndim == 3:
        query = query.mean(axis=1)
    fast_idx, ok = _topk_fast(query, memory_keys)
    top_idx = lax.cond(ok,
                       lambda: fast_idx,
                       lambda: _topk_indices(query, memory_keys))
    flat_idx = top_idx[:, :K].reshape(B * K)
    return _gather_mean()(memory_values, flat_idx)


# speedup vs baseline: 4.2907x; 1.0316x over previous
import functools

import jax
import jax.numpy as jnp
from jax import lax
from jax.experimental import pallas as pl
from jax.experimental.pallas import tpu as pltpu
from jax.experimental.pallas import tpu_sc as plsc

B = 1024
D = 32
CAP = 100000
K = 5
BLK = 8192
CAP_PAD = 106496
NBLK = CAP_PAD // BLK
BIGI = 2**30
BIGF = float(2**30)


def _topk_body_fast(q_ref, k_ref, oi_ref, om_ref, rv_ref, ri_ref, m3_ref):
    i = pl.program_id(0)

    @pl.when(i == 0)
    def _init():
        rv_ref[...] = jnp.full((B, 128), -jnp.inf, jnp.float32)
        ri_ref[...] = jnp.full((B, 128), BIGF, jnp.float32)
        m3_ref[...] = jnp.full((B, 128), -jnp.inf, jnp.float32)

    q = q_ref[...]
    qn = q / jnp.maximum(jnp.sqrt(jnp.sum(q * q, axis=1, keepdims=True)), 1e-12)
    kb = k_ref[...]
    kn = kb / jnp.maximum(jnp.sqrt(jnp.sum(kb * kb, axis=1, keepdims=True)), 1e-12)
    s = lax.dot_general(qn, kn, (((1,), (1,)), ((), ())),
                        preferred_element_type=jnp.float32)

    liota = lax.broadcasted_iota(jnp.int32, (B, 128), 1).astype(jnp.float32)
    oob_slice = (CAP - (NBLK - 1) * BLK) // 128
    fbase = (i * BLK).astype(jnp.float32)
    v1 = s1 = v2 = s2 = v3 = None
    for k in range(BLK // 128):
        x = s[:, k * 128:(k + 1) * 128]
        if k >= oob_slice:
            x = jnp.where(liota + (fbase + k * 128) < CAP, x, -jnp.inf)
        kf = jnp.float32(k)
        if k == 0:
            v1 = x
            s1 = jnp.zeros((B, 128), jnp.float32)
            v2 = jnp.full((B, 128), -jnp.inf, jnp.float32)
            s2 = jnp.zeros((B, 128), jnp.float32)
            v3 = jnp.full((B, 128), -jnp.inf, jnp.float32)
        else:
            gt1 = x > v1
            gt2 = x > v2
            gt3 = x > v3
            v3 = jnp.where(gt2, v2, jnp.where(gt3, x, v3))
            v2 = jnp.where(gt1, v1, jnp.where(gt2, x, v2))
            s2 = jnp.where(gt1, s1, jnp.where(gt2, kf, s2))
            v1 = jnp.where(gt1, x, v1)
            s1 = jnp.where(gt1, kf, s1)
    m3_ref[...] = jnp.maximum(
        m3_ref[...], jnp.max(v3, axis=1, keepdims=True))

    gb = liota + fbase
    c = jnp.concatenate([v1, v2, rv_ref[...]], axis=1)
    ci = jnp.concatenate([s1 * 128.0 + gb, s2 * 128.0 + gb, ri_ref[...]], axis=1)
    newv = []
    newi = []
    for _ in range(K):
        m = jnp.max(c, axis=1, keepdims=True)
        idx = jnp.min(jnp.where(c == m, ci, BIGF), axis=1, keepdims=True)
        newv.append(m)
        newi.append(idx)
        c = jnp.where(ci == idx, -jnp.inf, c)
    nv = jnp.concatenate(newv + [jnp.full((B, 128 - K), -jnp.inf, jnp.float32)], axis=1)
    ni = jnp.concatenate(newi + [jnp.full((B, 128 - K), BIGF, jnp.float32)], axis=1)
    rv_ref[...] = nv
    ri_ref[...] = ni

    @pl.when(i == NBLK - 1)
    def _emit():
        oi_ref[...] = ni.astype(jnp.int32)
        om_ref[...] = jnp.concatenate(
            [newv[K - 1], m3_ref[:, :1], jnp.zeros((B, 126), jnp.float32)], axis=1)


def _topk_fast(query, keys):
    oi, om = pl.pallas_call(
        _topk_body_fast,
        grid=(NBLK,),
        in_specs=[
            pl.BlockSpec((B, D), lambda i: (0, 0)),
            pl.BlockSpec((BLK, D), lambda i: (i, 0)),
        ],
        out_specs=[
            pl.BlockSpec((B, 128), lambda i: (0, 0)),
            pl.BlockSpec((B, 128), lambda i: (0, 0)),
        ],
        out_shape=[
            jax.ShapeDtypeStruct((B, 128), jnp.int32),
            jax.ShapeDtypeStruct((B, 128), jnp.float32),
        ],
        scratch_shapes=[
            pltpu.VMEM((B, 128), jnp.float32),
            pltpu.VMEM((B, 128), jnp.float32),
            pltpu.VMEM((B, 128), jnp.float32),
        ],
    )(query, keys)
    t5 = om[:, 0]
    m3 = om[:, 1]
    ok = jnp.all(m3 < t5)
    return oi, ok


def _topk_body(q_ref, k_ref, oi_ref, rv_ref, ri_ref):
    i = pl.program_id(0)

    @pl.when(i == 0)
    def _init():
        rv_ref[...] = jnp.full((B, 128), -jnp.inf, jnp.float32)
        ri_ref[...] = jnp.full((B, 128), BIGI, jnp.int32)

    q = q_ref[...]
    qn = q / jnp.maximum(jnp.sqrt(jnp.sum(q * q, axis=1, keepdims=True)), 1e-12)
    kb = k_ref[...]
    kn = kb / jnp.maximum(jnp.sqrt(jnp.sum(kb * kb, axis=1, keepdims=True)), 1e-12)
    s = lax.dot_general(qn, kn, (((1,), (1,)), ((), ())),
                        preferred_element_type=jnp.float32)
    giota = lax.broadcasted_iota(jnp.int32, (B, BLK), 1) + i * BLK
    s = jnp.where(giota < CAP, s, -jnp.inf)

    rv = rv_ref[...]
    ri = ri_ref[...]
    newv = []
    newi = []
    for _ in range(K):
        ms = jnp.max(s, axis=1, keepdims=True)
        mr = jnp.max(rv, axis=1, keepdims=True)
        m = jnp.maximum(ms, mr)
        is_ = jnp.min(jnp.where(s == m, giota, BIGI), axis=1, keepdims=True)
        ir_ = jnp.min(jnp.where(rv == m, ri, BIGI), axis=1, keepdims=True)
        idx = jnp.minimum(is_, ir_)
        newv.append(m)
        newi.append(idx)
        s = jnp.where(giota == idx, -jnp.inf, s)
        rv = jnp.where(ri == idx, -jnp.inf, rv)
    nv = jnp.concatenate(newv + [jnp.full((B, 128 - K), -jnp.inf, jnp.float32)], axis=1)
    ni = jnp.concatenate(newi + [jnp.full((B, 128 - K), BIGI, jnp.int32)], axis=1)
    rv_ref[...] = nv
    ri_ref[...] = ni
    oi_ref[...] = ni


def _topk_indices(query, keys):
    keys_p = jnp.pad(keys, ((0, CAP_PAD - CAP), (0, 0)))
    out = pl.pallas_call(
        _topk_body,
        grid=(NBLK,),
        in_specs=[
            pl.BlockSpec((B, D), lambda i: (0, 0)),
            pl.BlockSpec((BLK, D), lambda i: (i, 0)),
        ],
        out_specs=pl.BlockSpec((B, 128), lambda i: (0, 0)),
        out_shape=jax.ShapeDtypeStruct((B, 128), jnp.int32),
        scratch_shapes=[
            pltpu.VMEM((B, 128), jnp.float32),
            pltpu.VMEM((B, 128), jnp.int32),
        ],
    )(query, keys_p)
    return out


_NC = 2
_NS = 16
_NW = _NC * _NS
_QPW = B // _NW
_RPW = _QPW * K


def _gather_mean_body(table_hbm, idx_hbm, out_hbm, idx_v, rows_v, mean_v, sem):
    wid = lax.axis_index("s") * _NC + lax.axis_index("c")
    base = wid * _RPW
    pltpu.sync_copy(idx_hbm.at[pl.ds(base, _RPW)], idx_v)
    pltpu.async_copy(table_hbm.at[idx_v], rows_v, sem).wait()
    for q in range(_QPW):
        for h in range(D // 16):
            acc = rows_v[q * K + 0, pl.ds(h * 16, 16)]
            for j in range(1, K):
                acc = acc + rows_v[q * K + j, pl.ds(h * 16, 16)]
            mean_v[q, pl.ds(h * 16, 16)] = acc * jnp.float32(1.0 / K)
    pltpu.sync_copy(mean_v, out_hbm.at[pl.ds(wid * _QPW, _QPW)])


@functools.cache
def _gather_mean():
    return pl.kernel(
        _gather_mean_body,
        mesh=plsc.VectorSubcoreMesh(core_axis_name="c", subcore_axis_name="s"),
        out_type=jax.ShapeDtypeStruct((B, D), jnp.float32),
        scratch_types=[
            pltpu.VMEM((_RPW,), jnp.int32),
            pltpu.VMEM((_RPW, D), jnp.float32),
            pltpu.VMEM((_QPW, D), jnp.float32),
            pltpu.SemaphoreType.DMA,
        ],
        compiler_params=pltpu.CompilerParams(use_tc_tiling_on_sc=False),
    )


def kernel(query, memory_keys, memory_values):
    if query.ndim == 3:
        query = query.mean(axis=1)
    fast_idx, ok = _topk_fast(query, memory_keys)
    top_idx = lax.cond(ok,
                       lambda: fast_idx,
                       lambda: _topk_indices(query, memory_keys))
    flat_idx = top_idx[:, :K].reshape(B * K)
    return _gather_mean()(memory_values, flat_idx)


# speedup vs baseline: 4.5118x; 1.0515x over previous
import functools

import jax
import jax.numpy as jnp
from jax import lax
from jax.experimental import pallas as pl
from jax.experimental.pallas import tpu as pltpu
from jax.experimental.pallas import tpu_sc as plsc

B = 1024
D = 32
CAP = 100000
K = 5
BLK = 8192
CAP_PAD = 106496
NBLK = CAP_PAD // BLK
BIGI = 2**30
BIGF = float(2**30)


def _topk_body_fast(q_ref, k_ref, oi_ref, om_ref, rv_ref, ri_ref, m3_ref):
    i = pl.program_id(0)

    @pl.when(i == 0)
    def _init():
        rv_ref[...] = jnp.full((B, 128), -jnp.inf, jnp.float32)
        ri_ref[...] = jnp.full((B, 128), BIGF, jnp.float32)
        m3_ref[...] = jnp.full((B, 128), -jnp.inf, jnp.float32)

    q = q_ref[...]
    qn = q / jnp.maximum(jnp.sqrt(jnp.sum(q * q, axis=1, keepdims=True)), 1e-12)
    kb = k_ref[...]
    kn = kb / jnp.maximum(jnp.sqrt(jnp.sum(kb * kb, axis=1, keepdims=True)), 1e-12)
    s = lax.dot_general(qn, kn, (((1,), (1,)), ((), ())),
                        preferred_element_type=jnp.float32)

    liota = lax.broadcasted_iota(jnp.int32, (B, 128), 1).astype(jnp.float32)
    oob_slice = (CAP - (NBLK - 1) * BLK) // 128
    fbase = (i * BLK).astype(jnp.float32)
    v1 = s1 = v2 = s2 = v3 = None
    for k in range(BLK // 128):
        x = s[:, k * 128:(k + 1) * 128]
        if k >= oob_slice:
            x = jnp.where(liota + (fbase + k * 128) < CAP, x, -jnp.inf)
        kf = jnp.float32(k)
        if k == 0:
            v1 = x
            s1 = jnp.zeros((B, 128), jnp.float32)
            v2 = jnp.full((B, 128), -jnp.inf, jnp.float32)
            s2 = jnp.zeros((B, 128), jnp.float32)
            v3 = jnp.full((B, 128), -jnp.inf, jnp.float32)
        else:
            gt1 = x > v1
            gt2 = x > v2
            v3 = jnp.where(gt2, v2, jnp.maximum(v3, x))
            v2 = jnp.minimum(v1, jnp.maximum(x, v2))
            s2 = jnp.where(gt1, s1, jnp.where(gt2, kf, s2))
            v1 = jnp.maximum(x, v1)
            s1 = jnp.where(gt1, kf, s1)
    m3_ref[...] = jnp.maximum(
        m3_ref[...], jnp.max(v3, axis=1, keepdims=True))

    gb = liota + fbase
    c = jnp.concatenate([v1, v2, rv_ref[...]], axis=1)
    ci = jnp.concatenate([s1 * 128.0 + gb, s2 * 128.0 + gb, ri_ref[...]], axis=1)
    newv = []
    newi = []
    for _ in range(K):
        m = jnp.max(c, axis=1, keepdims=True)
        idx = jnp.min(jnp.where(c == m, ci, BIGF), axis=1, keepdims=True)
        newv.append(m)
        newi.append(idx)
        c = jnp.where(ci == idx, -jnp.inf, c)
    nv = jnp.concatenate(newv + [jnp.full((B, 128 - K), -jnp.inf, jnp.float32)], axis=1)
    ni = jnp.concatenate(newi + [jnp.full((B, 128 - K), BIGF, jnp.float32)], axis=1)
    rv_ref[...] = nv
    ri_ref[...] = ni

    @pl.when(i == NBLK - 1)
    def _emit():
        oi_ref[...] = ni.astype(jnp.int32)
        om_ref[...] = jnp.concatenate(
            [newv[K - 1], m3_ref[:, :1], jnp.zeros((B, 126), jnp.float32)], axis=1)


def _topk_fast(query, keys):
    oi, om = pl.pallas_call(
        _topk_body_fast,
        grid=(NBLK,),
        in_specs=[
            pl.BlockSpec((B, D), lambda i: (0, 0)),
            pl.BlockSpec((BLK, D), lambda i: (i, 0)),
        ],
        out_specs=[
            pl.BlockSpec((B, 128), lambda i: (0, 0)),
            pl.BlockSpec((B, 128), lambda i: (0, 0)),
        ],
        out_shape=[
            jax.ShapeDtypeStruct((B, 128), jnp.int32),
            jax.ShapeDtypeStruct((B, 128), jnp.float32),
        ],
        scratch_shapes=[
            pltpu.VMEM((B, 128), jnp.float32),
            pltpu.VMEM((B, 128), jnp.float32),
            pltpu.VMEM((B, 128), jnp.float32),
        ],
    )(query, keys)
    t5 = om[:, 0]
    m3 = om[:, 1]
    ok = jnp.all(m3 < t5)
    return oi, ok


def _topk_body(q_ref, k_ref, oi_ref, rv_ref, ri_ref):
    i = pl.program_id(0)

    @pl.when(i == 0)
    def _init():
        rv_ref[...] = jnp.full((B, 128), -jnp.inf, jnp.float32)
        ri_ref[...] = jnp.full((B, 128), BIGI, jnp.int32)

    q = q_ref[...]
    qn = q / jnp.maximum(jnp.sqrt(jnp.sum(q * q, axis=1, keepdims=True)), 1e-12)
    kb = k_ref[...]
    kn = kb / jnp.maximum(jnp.sqrt(jnp.sum(kb * kb, axis=1, keepdims=True)), 1e-12)
    s = lax.dot_general(qn, kn, (((1,), (1,)), ((), ())),
                        preferred_element_type=jnp.float32)
    giota = lax.broadcasted_iota(jnp.int32, (B, BLK), 1) + i * BLK
    s = jnp.where(giota < CAP, s, -jnp.inf)

    rv = rv_ref[...]
    ri = ri_ref[...]
    newv = []
    newi = []
    for _ in range(K):
        ms = jnp.max(s, axis=1, keepdims=True)
        mr = jnp.max(rv, axis=1, keepdims=True)
        m = jnp.maximum(ms, mr)
        is_ = jnp.min(jnp.where(s == m, giota, BIGI), axis=1, keepdims=True)
        ir_ = jnp.min(jnp.where(rv == m, ri, BIGI), axis=1, keepdims=True)
        idx = jnp.minimum(is_, ir_)
        newv.append(m)
        newi.append(idx)
        s = jnp.where(giota == idx, -jnp.inf, s)
        rv = jnp.where(ri == idx, -jnp.inf, rv)
    nv = jnp.concatenate(newv + [jnp.full((B, 128 - K), -jnp.inf, jnp.float32)], axis=1)
    ni = jnp.concatenate(newi + [jnp.full((B, 128 - K), BIGI, jnp.int32)], axis=1)
    rv_ref[...] = nv
    ri_ref[...] = ni
    oi_ref[...] = ni


def _topk_indices(query, keys):
    keys_p = jnp.pad(keys, ((0, CAP_PAD - CAP), (0, 0)))
    out = pl.pallas_call(
        _topk_body,
        grid=(NBLK,),
        in_specs=[
            pl.BlockSpec((B, D), lambda i: (0, 0)),
            pl.BlockSpec((BLK, D), lambda i: (i, 0)),
        ],
        out_specs=pl.BlockSpec((B, 128), lambda i: (0, 0)),
        out_shape=jax.ShapeDtypeStruct((B, 128), jnp.int32),
        scratch_shapes=[
            pltpu.VMEM((B, 128), jnp.float32),
            pltpu.VMEM((B, 128), jnp.int32),
        ],
    )(query, keys_p)
    return out


_NC = 2
_NS = 16
_NW = _NC * _NS
_QPW = B // _NW
_RPW = _QPW * K


def _gather_mean_body(table_hbm, idx_hbm, out_hbm, idx_v, rows_v, mean_v, sem):
    wid = lax.axis_index("s") * _NC + lax.axis_index("c")
    base = wid * _RPW
    pltpu.sync_copy(idx_hbm.at[pl.ds(base, _RPW)], idx_v)
    pltpu.async_copy(table_hbm.at[idx_v], rows_v, sem).wait()
    for q in range(_QPW):
        for h in range(D // 16):
            acc = rows_v[q * K + 0, pl.ds(h * 16, 16)]
            for j in range(1, K):
                acc = acc + rows_v[q * K + j, pl.ds(h * 16, 16)]
            mean_v[q, pl.ds(h * 16, 16)] = acc * jnp.float32(1.0 / K)
    pltpu.sync_copy(mean_v, out_hbm.at[pl.ds(wid * _QPW, _QPW)])


@functools.cache
def _gather_mean():
    return pl.kernel(
        _gather_mean_body,
        mesh=plsc.VectorSubcoreMesh(core_axis_name="c", subcore_axis_name="s"),
        out_type=jax.ShapeDtypeStruct((B, D), jnp.float32),
        scratch_types=[
            pltpu.VMEM((_RPW,), jnp.int32),
            pltpu.VMEM((_RPW, D), jnp.float32),
            pltpu.VMEM((_QPW, D), jnp.float32),
            pltpu.SemaphoreType.DMA,
        ],
        compiler_params=pltpu.CompilerParams(use_tc_tiling_on_sc=False),
    )


def kernel(query, memory_keys, memory_values):
    if query.ndim == 3:
        query = query.mean(axis=1)
    fast_idx, ok = _topk_fast(query, memory_keys)
    top_idx = lax.cond(ok,
                       lambda: fast_idx,
                       lambda: _topk_indices(query, memory_keys))
    flat_idx = top_idx[:, :K].reshape(B * K)
    return _gather_mean()(memory_values, flat_idx)


# speedup vs baseline: 4.5153x; 1.0008x over previous
import functools

import jax
import jax.numpy as jnp
from jax import lax
from jax.experimental import pallas as pl
from jax.experimental.pallas import tpu as pltpu
from jax.experimental.pallas import tpu_sc as plsc

B = 1024
D = 32
CAP = 100000
K = 5
BLK = 8192
CAP_PAD = 106496
NBLK = CAP_PAD // BLK
BIGI = 2**30
BIGF = float(2**30)


def _topk_body_fast(q_ref, k_ref, oi_ref, om_ref, rv_ref, ri_ref, m3_ref):
    i = pl.program_id(0)

    @pl.when(i == 0)
    def _init():
        rv_ref[...] = jnp.full((B, 128), -jnp.inf, jnp.float32)
        ri_ref[...] = jnp.full((B, 128), BIGF, jnp.float32)
        m3_ref[...] = jnp.full((B, 128), -jnp.inf, jnp.float32)

    q = q_ref[...]
    qn = q / jnp.maximum(jnp.sqrt(jnp.sum(q * q, axis=1, keepdims=True)), 1e-12)
    kb = k_ref[...]
    kn = kb / jnp.maximum(jnp.sqrt(jnp.sum(kb * kb, axis=1, keepdims=True)), 1e-12)
    s = lax.dot_general(qn, kn, (((1,), (1,)), ((), ())),
                        preferred_element_type=jnp.float32)

    liota = lax.broadcasted_iota(jnp.int32, (B, 128), 1).astype(jnp.float32)
    oob_slice = (CAP - (NBLK - 1) * BLK) // 128
    fbase = (i * BLK).astype(jnp.float32)
    v1 = s1 = v2 = s2 = v3 = None
    for k in range(BLK // 128):
        x = s[:, k * 128:(k + 1) * 128]
        if k >= oob_slice:
            x = jnp.where(liota + (fbase + k * 128) < CAP, x, -jnp.inf)
        kf = jnp.float32(k)
        if k == 0:
            v1 = x
            s1 = jnp.zeros((B, 128), jnp.float32)
            v2 = jnp.full((B, 128), -jnp.inf, jnp.float32)
            s2 = jnp.zeros((B, 128), jnp.float32)
            v3 = jnp.full((B, 128), -jnp.inf, jnp.float32)
        else:
            gt1 = x > v1
            gt2 = x > v2
            v3 = jnp.where(gt2, v2, jnp.maximum(v3, x))
            v2 = jnp.minimum(v1, jnp.maximum(x, v2))
            s2 = jnp.where(gt1, s1, jnp.where(gt2, kf, s2))
            v1 = jnp.maximum(x, v1)
            s1 = jnp.where(gt1, kf, s1)
    m3_ref[...] = jnp.maximum(
        m3_ref[...], jnp.max(v3, axis=1, keepdims=True))

    gb = liota + fbase
    c = jnp.concatenate([v1, v2, rv_ref[...]], axis=1)
    ci = jnp.concatenate([s1 * 128.0 + gb, s2 * 128.0 + gb, ri_ref[...]], axis=1)
    newv = []
    newi = []
    for _ in range(K):
        m = jnp.max(c, axis=1, keepdims=True)
        idx = jnp.min(jnp.where(c == m, ci, BIGF), axis=1, keepdims=True)
        newv.append(m)
        newi.append(idx)
        c = jnp.where(ci == idx, -jnp.inf, c)
    nv = jnp.concatenate(newv + [jnp.full((B, 128 - K), -jnp.inf, jnp.float32)], axis=1)
    ni = jnp.concatenate(newi + [jnp.full((B, 128 - K), BIGF, jnp.float32)], axis=1)
    rv_ref[...] = nv
    ri_ref[...] = ni

    @pl.when(i == NBLK - 1)
    def _emit():
        oi_ref[...] = ni.astype(jnp.int32)
        flag = jnp.max(m3_ref[:, :1] - newv[K - 1])
        om_ref[...] = jnp.full((8, 128), flag, jnp.float32)


def _topk_fast(query, keys):
    oi, om = pl.pallas_call(
        _topk_body_fast,
        grid=(NBLK,),
        in_specs=[
            pl.BlockSpec((B, D), lambda i: (0, 0)),
            pl.BlockSpec((BLK, D), lambda i: (i, 0)),
        ],
        out_specs=[
            pl.BlockSpec((B, 128), lambda i: (0, 0)),
            pl.BlockSpec((8, 128), lambda i: (0, 0)),
        ],
        out_shape=[
            jax.ShapeDtypeStruct((B, 128), jnp.int32),
            jax.ShapeDtypeStruct((8, 128), jnp.float32),
        ],
        scratch_shapes=[
            pltpu.VMEM((B, 128), jnp.float32),
            pltpu.VMEM((B, 128), jnp.float32),
            pltpu.VMEM((B, 128), jnp.float32),
        ],
    )(query, keys)
    ok = om[0, 0] < 0.0
    return oi, ok


def _topk_body(q_ref, k_ref, oi_ref, rv_ref, ri_ref):
    i = pl.program_id(0)

    @pl.when(i == 0)
    def _init():
        rv_ref[...] = jnp.full((B, 128), -jnp.inf, jnp.float32)
        ri_ref[...] = jnp.full((B, 128), BIGI, jnp.int32)

    q = q_ref[...]
    qn = q / jnp.maximum(jnp.sqrt(jnp.sum(q * q, axis=1, keepdims=True)), 1e-12)
    kb = k_ref[...]
    kn = kb / jnp.maximum(jnp.sqrt(jnp.sum(kb * kb, axis=1, keepdims=True)), 1e-12)
    s = lax.dot_general(qn, kn, (((1,), (1,)), ((), ())),
                        preferred_element_type=jnp.float32)
    giota = lax.broadcasted_iota(jnp.int32, (B, BLK), 1) + i * BLK
    s = jnp.where(giota < CAP, s, -jnp.inf)

    rv = rv_ref[...]
    ri = ri_ref[...]
    newv = []
    newi = []
    for _ in range(K):
        ms = jnp.max(s, axis=1, keepdims=True)
        mr = jnp.max(rv, axis=1, keepdims=True)
        m = jnp.maximum(ms, mr)
        is_ = jnp.min(jnp.where(s == m, giota, BIGI), axis=1, keepdims=True)
        ir_ = jnp.min(jnp.where(rv == m, ri, BIGI), axis=1, keepdims=True)
        idx = jnp.minimum(is_, ir_)
        newv.append(m)
        newi.append(idx)
        s = jnp.where(giota == idx, -jnp.inf, s)
        rv = jnp.where(ri == idx, -jnp.inf, rv)
    nv = jnp.concatenate(newv + [jnp.full((B, 128 - K), -jnp.inf, jnp.float32)], axis=1)
    ni = jnp.concatenate(newi + [jnp.full((B, 128 - K), BIGI, jnp.int32)], axis=1)
    rv_ref[...] = nv
    ri_ref[...] = ni
    oi_ref[...] = ni


def _topk_indices(query, keys):
    keys_p = jnp.pad(keys, ((0, CAP_PAD - CAP), (0, 0)))
    out = pl.pallas_call(
        _topk_body,
        grid=(NBLK,),
        in_specs=[
            pl.BlockSpec((B, D), lambda i: (0, 0)),
            pl.BlockSpec((BLK, D), lambda i: (i, 0)),
        ],
        out_specs=pl.BlockSpec((B, 128), lambda i: (0, 0)),
        out_shape=jax.ShapeDtypeStruct((B, 128), jnp.int32),
        scratch_shapes=[
            pltpu.VMEM((B, 128), jnp.float32),
            pltpu.VMEM((B, 128), jnp.int32),
        ],
    )(query, keys_p)
    return out


_NC = 2
_NS = 16
_NW = _NC * _NS
_QPW = B // _NW
_RPW = _QPW * K


def _gather_mean_body(table_hbm, idx_hbm, out_hbm, idx_v, rows_v, mean_v, sem):
    wid = lax.axis_index("s") * _NC + lax.axis_index("c")
    base = wid * _RPW
    pltpu.sync_copy(idx_hbm.at[pl.ds(base, _RPW)], idx_v)
    pltpu.async_copy(table_hbm.at[idx_v], rows_v, sem).wait()
    for q in range(_QPW):
        for h in range(D // 16):
            acc = rows_v[q * K + 0, pl.ds(h * 16, 16)]
            for j in range(1, K):
                acc = acc + rows_v[q * K + j, pl.ds(h * 16, 16)]
            mean_v[q, pl.ds(h * 16, 16)] = acc * jnp.float32(1.0 / K)
    pltpu.sync_copy(mean_v, out_hbm.at[pl.ds(wid * _QPW, _QPW)])


@functools.cache
def _gather_mean():
    return pl.kernel(
        _gather_mean_body,
        mesh=plsc.VectorSubcoreMesh(core_axis_name="c", subcore_axis_name="s"),
        out_type=jax.ShapeDtypeStruct((B, D), jnp.float32),
        scratch_types=[
            pltpu.VMEM((_RPW,), jnp.int32),
            pltpu.VMEM((_RPW, D), jnp.float32),
            pltpu.VMEM((_QPW, D), jnp.float32),
            pltpu.SemaphoreType.DMA,
        ],
        compiler_params=pltpu.CompilerParams(use_tc_tiling_on_sc=False),
    )


def kernel(query, memory_keys, memory_values):
    if query.ndim == 3:
        query = query.mean(axis=1)
    fast_idx, ok = _topk_fast(query, memory_keys)
    top_idx = lax.cond(ok,
                       lambda: fast_idx,
                       lambda: _topk_indices(query, memory_keys))
    flat_idx = top_idx[:, :K].reshape(B * K)
    return _gather_mean()(memory_values, flat_idx)
